# initial kernel scaffold (unmeasured)
import functools

import jax
import jax.numpy as jnp
from jax import lax
from jax.experimental import pallas as pl
from jax.experimental.pallas import tpu as pltpu

N_DEV = 4
M_PER = 2048
N_COLS = 4096
N_HALF = 2048


def _gelu(y):
    c = 0.7978845608028654
    return 0.5 * y * (1.0 + jnp.tanh(c * (y + 0.044715 * y * y * y)))


def _rs_body(p_ref, out_ref, comm, stage, send_sems, recv_sems, copy_sem,
             out_sem):
    d = lax.axis_index("i")
    left = lax.rem(d - 1 + N_DEV, N_DEV)
    right = lax.rem(d + 1, N_DEV)

    barrier_sem = pltpu.get_barrier_semaphore()
    for nbr in (left, right):
        pl.semaphore_signal(
            barrier_sem, inc=1,
            device_id=(nbr,), device_id_type=pl.DeviceIdType.MESH,
        )
    pl.semaphore_wait(barrier_sem, 2)

    for half in range(2):
        cols = pl.ds(half * N_HALF, N_HALF)

        c0 = lax.rem(d - 1 + N_DEV, N_DEV)
        init = pltpu.make_async_copy(
            p_ref.at[pl.ds(c0 * M_PER, M_PER), cols], comm.at[0], copy_sem)
        init.start()
        init.wait()

        for s in range(N_DEV - 1):
            send_slot = s % 2
            recv_slot = (s + 1) % 2
            rdma = pltpu.make_async_remote_copy(
                src_ref=comm.at[send_slot],
                dst_ref=comm.at[recv_slot],
                send_sem=send_sems.at[half, s],
                recv_sem=recv_sems.at[half, s],
                device_id=(right,),
                device_id_type=pl.DeviceIdType.MESH,
            )
            rdma.start()

            c_recv = lax.rem(d - 2 - s + 2 * N_DEV, N_DEV)
            cp = pltpu.make_async_copy(
                p_ref.at[pl.ds(c_recv * M_PER, M_PER), cols], stage, copy_sem)
            cp.start()

            rdma.wait()
            cp.wait()

            if s < N_DEV - 2:
                comm[recv_slot] = comm[recv_slot] + stage[...]
            else:
                stage[...] = _gelu(comm[recv_slot] + stage[...])
                out_cp = pltpu.make_async_copy(
                    stage, out_ref.at[:, cols], out_sem)
                out_cp.start()
                out_cp.wait()


def _reduce_scatter_gelu(p):
    return pl.pallas_call(
        _rs_body,
        out_shape=jax.ShapeDtypeStruct((M_PER, N_COLS), jnp.float32),
        in_specs=[pl.BlockSpec(memory_space=pltpu.MemorySpace.HBM)],
        out_specs=pl.BlockSpec(memory_space=pltpu.MemorySpace.HBM),
        scratch_shapes=[
            pltpu.VMEM((2, M_PER, N_HALF), jnp.float32),
            pltpu.VMEM((M_PER, N_HALF), jnp.float32),
            pltpu.SemaphoreType.DMA((2, N_DEV - 1)),
            pltpu.SemaphoreType.DMA((2, N_DEV - 1)),
            pltpu.SemaphoreType.DMA,
            pltpu.SemaphoreType.DMA,
        ],
        compiler_params=pltpu.CompilerParams(
            collective_id=0,
            has_side_effects=True,
        ),
    )(p)


def kernel(x, w_mat):
    xb = x.astype(jnp.bfloat16)
    wb = w_mat.astype(jnp.bfloat16)
    p = jnp.dot(xb, wb, preferred_element_type=jnp.float32)
    return _reduce_scatter_gelu(p)


# baseline (device time: 1359947 ns/iter reference)
import functools

import jax
import jax.numpy as jnp
from jax import lax
from jax.experimental import pallas as pl
from jax.experimental.pallas import tpu as pltpu

N_DEV = 4
M_PER = 2048
N_COLS = 4096
N_HALF = 2048


def _gelu(y):
    c = 0.7978845608028654
    return 0.5 * y * (1.0 + jnp.tanh(c * (y + 0.044715 * y * y * y)))


def _rs_body(p_ref, out_ref, comm, stage, send_sems, recv_sems, copy_sem,
             out_sem):
    d = lax.axis_index("i")
    left = lax.rem(d - 1 + N_DEV, N_DEV)
    right = lax.rem(d + 1, N_DEV)

    barrier_sem = pltpu.get_barrier_semaphore()
    for nbr in (left, right):
        pl.semaphore_signal(
            barrier_sem, inc=1,
            device_id=(nbr,), device_id_type=pl.DeviceIdType.MESH,
        )
    pl.semaphore_wait(barrier_sem, 2)

    for half in range(2):
        cols = pl.ds(half * N_HALF, N_HALF)

        c0 = lax.rem(d - 1 + N_DEV, N_DEV)
        init = pltpu.make_async_copy(
            p_ref.at[pl.ds(c0 * M_PER, M_PER), cols], comm.at[0], copy_sem)
        init.start()
        init.wait()

        for s in range(N_DEV - 1):
            send_slot = s % 2
            recv_slot = (s + 1) % 2
            rdma = pltpu.make_async_remote_copy(
                src_ref=comm.at[send_slot],
                dst_ref=comm.at[recv_slot],
                send_sem=send_sems.at[half, s],
                recv_sem=recv_sems.at[half, s],
                device_id=(right,),
                device_id_type=pl.DeviceIdType.MESH,
            )
            rdma.start()

            c_recv = lax.rem(d - 2 - s + 2 * N_DEV, N_DEV)
            cp = pltpu.make_async_copy(
                p_ref.at[pl.ds(c_recv * M_PER, M_PER), cols], stage, copy_sem)
            cp.start()

            rdma.wait()
            cp.wait()

            if s < N_DEV - 2:
                comm[recv_slot] = comm[recv_slot] + stage[...]
            else:
                stage[...] = _gelu(comm[recv_slot] + stage[...])
                out_cp = pltpu.make_async_copy(
                    stage, out_ref.at[:, cols], out_sem)
                out_cp.start()
                out_cp.wait()


def _reduce_scatter_gelu(p):
    return pl.pallas_call(
        _rs_body,
        out_shape=jax.ShapeDtypeStruct((M_PER, N_COLS), jnp.float32),
        in_specs=[pl.BlockSpec(memory_space=pltpu.MemorySpace.HBM)],
        out_specs=pl.BlockSpec(memory_space=pltpu.MemorySpace.HBM),
        scratch_shapes=[
            pltpu.VMEM((2, M_PER, N_HALF), jnp.float32),
            pltpu.VMEM((M_PER, N_HALF), jnp.float32),
            pltpu.SemaphoreType.DMA((2, N_DEV - 1)),
            pltpu.SemaphoreType.DMA((2, N_DEV - 1)),
            pltpu.SemaphoreType.DMA,
            pltpu.SemaphoreType.DMA,
        ],
        compiler_params=pltpu.CompilerParams(
            collective_id=0,
            has_side_effects=True,
            vmem_limit_bytes=60 * 1024 * 1024,
        ),
    )(p)


def kernel(x, w_mat):
    xb = x.astype(jnp.bfloat16)
    wb = w_mat.astype(jnp.bfloat16)
    p = jnp.dot(xb, wb, preferred_element_type=jnp.float32)
    return _reduce_scatter_gelu(p)


# device time: 568780 ns/iter; 2.3910x vs baseline; 2.3910x over previous
import jax
import jax.numpy as jnp
from jax import lax
from jax.experimental import pallas as pl
from jax.experimental.pallas import tpu as pltpu

N_DEV = 4
M_PER = 2048
N_COLS = 4096
N_HALF = 2048
TILE = 512
N_TILES = M_PER // TILE


def _gelu(y):
    c = 0.7978845608028654
    return 0.5 * y * (1.0 + jnp.tanh(c * (y + 0.044715 * y * y * y)))


def _rs_body(p_ref, out_ref, comms, pstage, ostage,
             send_sems, recv_sems, p_sems, o_sems):
    d = lax.axis_index("i")
    left = lax.rem(d + N_DEV - 1, N_DEV)
    right = lax.rem(d + 1, N_DEV)

    barrier_sem = pltpu.get_barrier_semaphore()
    for nbr in (left, right):
        pl.semaphore_signal(
            barrier_sem, inc=1,
            device_id=(nbr,), device_id_type=pl.DeviceIdType.MESH,
        )
    pl.semaphore_wait(barrier_sem, 2)

    def p_tile_copy(dir_idx, c, t):
        rows = pl.ds(c * M_PER + t * TILE, TILE)
        cols = pl.ds(dir_idx * N_HALF, N_HALF)
        return pltpu.make_async_copy(
            p_ref.at[rows, cols], pstage.at[dir_idx, t % 2],
            p_sems.at[dir_idx, t % 2])

    def fill_or_accum(dir_idx, slot, c, init):
        cps = [p_tile_copy(dir_idx, c, t) for t in range(N_TILES)]
        cps[0].start()
        for t in range(N_TILES):
            if t + 1 < N_TILES:
                cps[t + 1].start()
            cps[t].wait()
            rows = pl.ds(t * TILE, TILE)
            pt = pstage[dir_idx, t % 2]
            if init:
                comms[dir_idx, slot, rows, :] = pt.astype(jnp.bfloat16)
            else:
                acc = comms[dir_idx, slot, rows, :].astype(jnp.float32) + pt
                comms[dir_idx, slot, rows, :] = acc.astype(jnp.bfloat16)

    def epilogue(dir_idx, slot, c):
        cps = [p_tile_copy(dir_idx, c, t) for t in range(N_TILES)]
        cps[0].start()
        outs = []
        for t in range(N_TILES):
            if t + 1 < N_TILES:
                cps[t + 1].start()
            cps[t].wait()
            rows = pl.ds(t * TILE, TILE)
            if t >= 2:
                outs[t - 2].wait()
            acc = comms[dir_idx, slot, rows, :].astype(jnp.float32) \
                + pstage[dir_idx, t % 2]
            ostage[t % 2] = _gelu(acc)
            ocp = pltpu.make_async_copy(
                ostage.at[t % 2],
                out_ref.at[rows, pl.ds(dir_idx * N_HALF, N_HALF)],
                o_sems.at[t % 2])
            ocp.start()
            outs.append(ocp)
        outs[-2].wait()
        outs[-1].wait()

    def chunk_r(k):
        return lax.rem(d + 2 * N_DEV - k, N_DEV)

    def chunk_l(k):
        return lax.rem(d + k, N_DEV)

    fill_or_accum(0, 0, chunk_r(1), init=True)
    fill_or_accum(1, 0, chunk_l(1), init=True)

    for s in range(N_DEV - 1):
        send_slot = s % 2
        recv_slot = (s + 1) % 2
        rdmas = []
        for dir_idx, tgt in ((0, right), (1, left)):
            rdma = pltpu.make_async_remote_copy(
                src_ref=comms.at[dir_idx, send_slot],
                dst_ref=comms.at[dir_idx, recv_slot],
                send_sem=send_sems.at[dir_idx, s],
                recv_sem=recv_sems.at[dir_idx, s],
                device_id=(tgt,),
                device_id_type=pl.DeviceIdType.MESH,
            )
            rdma.start()
            rdmas.append(rdma)

        last = s == N_DEV - 2
        for dir_idx, c in ((0, chunk_r(2 + s)), (1, chunk_l(2 + s))):
            rdmas[dir_idx].wait_recv()
            if last:
                epilogue(dir_idx, recv_slot, c)
            else:
                fill_or_accum(dir_idx, recv_slot, c, init=False)
        for rdma in rdmas:
            rdma.wait_send()


def _reduce_scatter_gelu(p):
    return pl.pallas_call(
        _rs_body,
        out_shape=jax.ShapeDtypeStruct((M_PER, N_COLS), jnp.float32),
        in_specs=[pl.BlockSpec(memory_space=pltpu.MemorySpace.HBM)],
        out_specs=pl.BlockSpec(memory_space=pltpu.MemorySpace.HBM),
        scratch_shapes=[
            pltpu.VMEM((2, 2, M_PER, N_HALF), jnp.bfloat16),
            pltpu.VMEM((2, 2, TILE, N_HALF), jnp.float32),
            pltpu.VMEM((2, TILE, N_HALF), jnp.float32),
            pltpu.SemaphoreType.DMA((2, N_DEV - 1)),
            pltpu.SemaphoreType.DMA((2, N_DEV - 1)),
            pltpu.SemaphoreType.DMA((2, 2)),
            pltpu.SemaphoreType.DMA((2,)),
        ],
        compiler_params=pltpu.CompilerParams(
            collective_id=0,
            has_side_effects=True,
            vmem_limit_bytes=62 * 1024 * 1024,
        ),
    )(p)


def kernel(x, w_mat):
    xb = x.astype(jnp.bfloat16)
    wb = w_mat.astype(jnp.bfloat16)
    p = jnp.dot(xb, wb, preferred_element_type=jnp.float32)
    return _reduce_scatter_gelu(p)


# device time: 541712 ns/iter; 2.5105x vs baseline; 1.0500x over previous
import jax

jax.config.update("jax_compilation_cache_dir",
                  "/tmp/scband_problems/jax_cache")
jax.config.update("jax_persistent_cache_min_compile_time_secs", 1.0)

import jax.numpy as jnp
from jax import lax
from jax.experimental import pallas as pl
from jax.experimental.pallas import tpu as pltpu

N_DEV = 4
M_PER = 2048
N_COLS = 4096
N_HALF = 2048
K = 2048
TILE = 256
N_TILES = M_PER // TILE


def _gelu(y):
    c = 0.7978845608028654
    return 0.5 * y * (1.0 + jnp.tanh(c * (y + 0.044715 * y * y * y)))


def _body(x_ref, w0_ref, w1_ref, out_ref, comms, xstage, ostage,
          send_sems, recv_sems, x_sems, o_sems):
    d = lax.axis_index("i")
    left = lax.rem(d + N_DEV - 1, N_DEV)
    right = lax.rem(d + 1, N_DEV)

    barrier_sem = pltpu.get_barrier_semaphore()
    for nbr in (left, right):
        pl.semaphore_signal(
            barrier_sem, inc=1,
            device_id=(nbr,), device_id_type=pl.DeviceIdType.MESH,
        )
    pl.semaphore_wait(barrier_sem, 2)

    def x_tile_copy(dir_idx, c, t, par):
        rows = pl.ds(c * M_PER + t * TILE, TILE)
        return pltpu.make_async_copy(
            x_ref.at[rows, :], xstage.at[dir_idx, par],
            x_sems.at[dir_idx, par])

    def p_tile(dir_idx, par):
        w_half = w0_ref if dir_idx == 0 else w1_ref
        return jnp.dot(xstage[dir_idx, par], w_half[...],
                       preferred_element_type=jnp.float32)

    def fill_or_accum(dir_idx, slot, c, init):
        x_tile_copy(dir_idx, c, 0, 0).start()

        def body(i, carry):
            for par in (0, 1):
                t = 2 * i + par

                @pl.when(t + 1 < N_TILES)
                def _():
                    x_tile_copy(dir_idx, c, t + 1, 1 - par).start()

                x_tile_copy(dir_idx, c, t, par).wait()
                rows = pl.ds(t * TILE, TILE)
                pt = p_tile(dir_idx, par)
                if init:
                    comms[dir_idx, slot, rows, :] = pt.astype(jnp.bfloat16)
                else:
                    acc = comms[dir_idx, slot, rows, :].astype(jnp.float32) \
                        + pt
                    comms[dir_idx, slot, rows, :] = acc.astype(jnp.bfloat16)
            return carry

        lax.fori_loop(0, N_TILES // 2, body, 0)

    def epilogue(dir_idx, slot, c):
        x_tile_copy(dir_idx, c, 0, 0).start()
        ocols = pl.ds(dir_idx * N_HALF, N_HALF)

        def ocp(t, par):
            rows = pl.ds(t * TILE, TILE)
            return pltpu.make_async_copy(
                ostage.at[par], out_ref.at[rows, ocols], o_sems.at[par])

        def body(i, carry):
            for par in (0, 1):
                t = 2 * i + par

                @pl.when(t + 1 < N_TILES)
                def _():
                    x_tile_copy(dir_idx, c, t + 1, 1 - par).start()

                x_tile_copy(dir_idx, c, t, par).wait()

                @pl.when(t >= 2)
                def _():
                    ocp(t - 2, par).wait()

                rows = pl.ds(t * TILE, TILE)
                acc = comms[dir_idx, slot, rows, :].astype(jnp.float32) \
                    + p_tile(dir_idx, par)
                ostage[par] = _gelu(acc)
                ocp(t, par).start()
            return carry

        lax.fori_loop(0, N_TILES // 2, body, 0)
        ocp(N_TILES - 2, 0).wait()
        ocp(N_TILES - 1, 1).wait()

    def chunk_r(k):
        return lax.rem(d + 2 * N_DEV - k, N_DEV)

    def chunk_l(k):
        return lax.rem(d + k, N_DEV)

    fill_or_accum(0, 0, chunk_r(1), init=True)
    fill_or_accum(1, 0, chunk_l(1), init=True)

    for s in range(N_DEV - 1):
        send_slot = s % 2
        recv_slot = (s + 1) % 2
        rdmas = []
        for dir_idx, tgt in ((0, right), (1, left)):
            rdma = pltpu.make_async_remote_copy(
                src_ref=comms.at[dir_idx, send_slot],
                dst_ref=comms.at[dir_idx, recv_slot],
                send_sem=send_sems.at[dir_idx, s],
                recv_sem=recv_sems.at[dir_idx, s],
                device_id=(tgt,),
                device_id_type=pl.DeviceIdType.MESH,
            )
            rdma.start()
            rdmas.append(rdma)

        last = s == N_DEV - 2
        for dir_idx, c in ((0, chunk_r(2 + s)), (1, chunk_l(2 + s))):
            rdmas[dir_idx].wait_recv()
            if last:
                epilogue(dir_idx, recv_slot, c)
            else:
                fill_or_accum(dir_idx, recv_slot, c, init=False)
        for rdma in rdmas:
            rdma.wait_send()


def _fused_gemm_rs_gelu(xb, wb0, wb1):
    return pl.pallas_call(
        _body,
        out_shape=jax.ShapeDtypeStruct((M_PER, N_COLS), jnp.float32),
        in_specs=[
            pl.BlockSpec(memory_space=pltpu.MemorySpace.HBM),
            pl.BlockSpec(memory_space=pltpu.MemorySpace.VMEM),
            pl.BlockSpec(memory_space=pltpu.MemorySpace.VMEM),
        ],
        out_specs=pl.BlockSpec(memory_space=pltpu.MemorySpace.HBM),
        scratch_shapes=[
            pltpu.VMEM((2, 2, M_PER, N_HALF), jnp.bfloat16),
            pltpu.VMEM((2, 2, TILE, K), jnp.bfloat16),
            pltpu.VMEM((2, TILE, N_HALF), jnp.float32),
            pltpu.SemaphoreType.DMA((2, N_DEV - 1)),
            pltpu.SemaphoreType.DMA((2, N_DEV - 1)),
            pltpu.SemaphoreType.DMA((2, 2)),
            pltpu.SemaphoreType.DMA((2,)),
        ],
        compiler_params=pltpu.CompilerParams(
            collective_id=0,
            has_side_effects=True,
            vmem_limit_bytes=62 * 1024 * 1024,
        ),
    )(xb, wb0, wb1)


def kernel(x, w_mat):
    xb = x.astype(jnp.bfloat16)
    wb = w_mat.astype(jnp.bfloat16)
    return _fused_gemm_rs_gelu(xb, wb[:, :N_HALF], wb[:, N_HALF:])


# device time: 479199 ns/iter; 2.8380x vs baseline; 1.1305x over previous
import jax

jax.config.update("jax_compilation_cache_dir",
                  "/tmp/scband_problems/jax_cache")
jax.config.update("jax_persistent_cache_min_compile_time_secs", 1.0)

import jax.numpy as jnp
from jax import lax
from jax.experimental import pallas as pl
from jax.experimental.pallas import tpu as pltpu

N_DEV = 4
M_PER = 2048
N_COLS = 4096
N_HALF = 2048
K = 2048
TILE = 256
N_TILES = M_PER // TILE


def _gelu(y):
    c = 0.7978845608028654
    return 0.5 * y * (1.0 + jnp.tanh(c * (y + 0.044715 * y * y * y)))


def _body(x_ref, w0_ref, w1_ref, out_ref, comms, xstage, ostage,
          send_sems, recv_sems, x_sems, o_sems):
    d = lax.axis_index("i")
    left = lax.rem(d + N_DEV - 1, N_DEV)
    right = lax.rem(d + 1, N_DEV)

    barrier_sem = pltpu.get_barrier_semaphore()
    for nbr in (left, right):
        pl.semaphore_signal(
            barrier_sem, inc=1,
            device_id=(nbr,), device_id_type=pl.DeviceIdType.MESH,
        )
    pl.semaphore_wait(barrier_sem, 2)

    def x_tile_copy(dir_idx, c, t, par):
        rows = pl.ds(c * M_PER + t * TILE, TILE)
        return pltpu.make_async_copy(
            x_ref.at[rows, :], xstage.at[dir_idx, par],
            x_sems.at[dir_idx, par])

    def p_tile(dir_idx, par):
        w_half = w0_ref if dir_idx == 0 else w1_ref
        return jnp.dot(xstage[dir_idx, par], w_half[...],
                       preferred_element_type=jnp.float32)

    def fill_or_accum(dir_idx, slot, c, init):
        x_tile_copy(dir_idx, c, 0, 0).start()

        def body(i, carry):
            for par in (0, 1):
                t = 2 * i + par

                @pl.when(t + 1 < N_TILES)
                def _():
                    x_tile_copy(dir_idx, c, t + 1, 1 - par).start()

                x_tile_copy(dir_idx, c, t, par).wait()
                rows = pl.ds(t * TILE, TILE)
                pt = p_tile(dir_idx, par)
                if init:
                    comms[dir_idx, slot, rows, :] = pt.astype(jnp.bfloat16)
                else:
                    acc = comms[dir_idx, slot, rows, :].astype(jnp.float32) \
                        + pt
                    comms[dir_idx, slot, rows, :] = acc.astype(jnp.bfloat16)
            return carry

        lax.fori_loop(0, N_TILES // 2, body, 0)

    def epilogue(dir_idx, slot, c):
        x_tile_copy(dir_idx, c, 0, 0).start()
        ocols = pl.ds(dir_idx * N_HALF, N_HALF)

        def ocp(t, par):
            rows = pl.ds(t * TILE, TILE)
            return pltpu.make_async_copy(
                ostage.at[par], out_ref.at[rows, ocols], o_sems.at[par])

        def body(i, carry):
            for par in (0, 1):
                t = 2 * i + par

                @pl.when(t + 1 < N_TILES)
                def _():
                    x_tile_copy(dir_idx, c, t + 1, 1 - par).start()

                x_tile_copy(dir_idx, c, t, par).wait()

                @pl.when(t >= 2)
                def _():
                    ocp(t - 2, par).wait()

                rows = pl.ds(t * TILE, TILE)
                acc = comms[dir_idx, slot, rows, :].astype(jnp.float32) \
                    + p_tile(dir_idx, par)
                ostage[par] = _gelu(acc)
                ocp(t, par).start()
            return carry

        lax.fori_loop(0, N_TILES // 2, body, 0)
        ocp(N_TILES - 2, 0).wait()
        ocp(N_TILES - 1, 1).wait()

    def chunk_r(k):
        return lax.rem(d + 2 * N_DEV - k, N_DEV)

    def chunk_l(k):
        return lax.rem(d + k, N_DEV)

    def rdma_desc(dir_idx, s, tgt):
        return pltpu.make_async_remote_copy(
            src_ref=comms.at[dir_idx, s % 2],
            dst_ref=comms.at[dir_idx, (s + 1) % 2],
            send_sem=send_sems.at[dir_idx, s],
            recv_sem=recv_sems.at[dir_idx, s],
            device_id=(tgt,),
            device_id_type=pl.DeviceIdType.MESH,
        )

    tgts = (right, left)

    fill_or_accum(0, 0, chunk_r(1), init=True)
    rdma_desc(0, 0, right).start()
    fill_or_accum(1, 0, chunk_l(1), init=True)
    rdma_desc(1, 0, left).start()

    for s in range(N_DEV - 1):
        last = s == N_DEV - 2
        recv_slot = (s + 1) % 2
        for dir_idx in (0, 1):
            c = chunk_r(2 + s) if dir_idx == 0 else chunk_l(2 + s)
            desc = rdma_desc(dir_idx, s, tgts[dir_idx])
            desc.wait_recv()
            if last:
                epilogue(dir_idx, recv_slot, c)
            else:
                fill_or_accum(dir_idx, recv_slot, c, init=False)
                rdma_desc(dir_idx, s + 1, tgts[dir_idx]).start()
            desc.wait_send()


def _fused_gemm_rs_gelu(xb, wb0, wb1):
    return pl.pallas_call(
        _body,
        out_shape=jax.ShapeDtypeStruct((M_PER, N_COLS), jnp.float32),
        in_specs=[
            pl.BlockSpec(memory_space=pltpu.MemorySpace.HBM),
            pl.BlockSpec(memory_space=pltpu.MemorySpace.VMEM),
            pl.BlockSpec(memory_space=pltpu.MemorySpace.VMEM),
        ],
        out_specs=pl.BlockSpec(memory_space=pltpu.MemorySpace.HBM),
        scratch_shapes=[
            pltpu.VMEM((2, 2, M_PER, N_HALF), jnp.bfloat16),
            pltpu.VMEM((2, 2, TILE, K), jnp.bfloat16),
            pltpu.VMEM((2, TILE, N_HALF), jnp.float32),
            pltpu.SemaphoreType.DMA((2, N_DEV - 1)),
            pltpu.SemaphoreType.DMA((2, N_DEV - 1)),
            pltpu.SemaphoreType.DMA((2, 2)),
            pltpu.SemaphoreType.DMA((2,)),
        ],
        compiler_params=pltpu.CompilerParams(
            collective_id=0,
            has_side_effects=True,
            vmem_limit_bytes=62 * 1024 * 1024,
        ),
    )(xb, wb0, wb1)


def kernel(x, w_mat):
    xb = x.astype(jnp.bfloat16)
    wb = w_mat.astype(jnp.bfloat16)
    return _fused_gemm_rs_gelu(xb, wb[:, :N_HALF], wb[:, N_HALF:])
